# Initial kernel scaffold; baseline (speedup 1.0000x reference)
import functools

import jax
import jax.numpy as jnp
from jax import lax
from jax.experimental import pallas as pl
from jax.experimental.pallas import tpu as pltpu
from jax.experimental.pallas import tpu_sc as plsc

NPAD = 100352
NPT = NPAD // 16
CH = 128
K = 8
BN = 2048
_F32 = jnp.float32

_MESH = plsc.VectorSubcoreMesh(core_axis_name="c", subcore_axis_name="s")


def _deg_kernel(nbursts):
    @functools.partial(
        pl.kernel,
        out_type=jax.ShapeDtypeStruct((2 * NPAD,), _F32),
        mesh=_MESH,
        scratch_types=[
            pltpu.VMEM((K, CH), jnp.int32),
            pltpu.VMEM((K, CH), _F32),
            pltpu.VMEM_SHARED((NPAD,), _F32),
            pltpu.SemaphoreType.DMA,
        ],
    )
    def deg(dst2d, ones_h, zeros1, out, didx, ones_v, acc, sem):
        c = lax.axis_index("c")
        s = lax.axis_index("s")
        off = pl.multiple_of(s * NPT, 8)
        pltpu.sync_copy(zeros1, acc.at[pl.ds(off, NPT)])
        pltpu.sync_copy(ones_h, ones_v)
        plsc.subcore_barrier()
        row0 = (c * 16 + s) * (nbursts * K)

        def body(i, carry):
            rb = pl.multiple_of(row0 + i * K, 8)
            pltpu.sync_copy(dst2d.at[pl.ds(rb, K)], didx)
            descs = [
                pltpu.async_copy(ones_v.at[j], acc.at[didx.at[j]], sem, add=True)
                for j in range(K)
            ]
            for d in descs:
                d.wait()
            return carry

        lax.fori_loop(0, nbursts, body, 0)
        plsc.subcore_barrier()
        dst_off = pl.multiple_of(c * NPAD + off, 8)
        pltpu.sync_copy(acc.at[pl.ds(off, NPT)], out.at[pl.ds(dst_off, NPT)])

    return deg


def _agg_kernel(nbursts):
    @functools.partial(
        pl.kernel,
        out_type=jax.ShapeDtypeStruct((2 * NPAD, 16), _F32),
        mesh=_MESH,
        scratch_types=[
            pltpu.VMEM((K, CH), jnp.int32),
            pltpu.VMEM((K, CH), jnp.int32),
            pltpu.VMEM((K, CH, 16), _F32),
            pltpu.VMEM_SHARED((NPAD, 16), _F32),
            pltpu.SemaphoreType.DMA,
            pltpu.SemaphoreType.DMA,
        ],
    )
    def agg(table, src2d, dst2d, zrows, out, sidx, didx, rows, acc, sem_g, sem_s):
        c = lax.axis_index("c")
        s = lax.axis_index("s")
        off = pl.multiple_of(s * NPT, 8)
        pltpu.sync_copy(zrows, acc.at[pl.ds(off, NPT)])
        plsc.subcore_barrier()
        row0 = (c * 16 + s) * (nbursts * K)

        def body(i, carry):
            rb = pl.multiple_of(row0 + i * K, 8)
            pltpu.sync_copy(src2d.at[pl.ds(rb, K)], sidx)
            pltpu.sync_copy(dst2d.at[pl.ds(rb, K)], didx)
            gd = [
                pltpu.async_copy(table.at[sidx.at[j]], rows.at[j], sem_g)
                for j in range(K)
            ]
            for d in gd:
                d.wait()
            sd = [
                pltpu.async_copy(rows.at[j], acc.at[didx.at[j]], sem_s, add=True)
                for j in range(K)
            ]
            for d in sd:
                d.wait()
            return carry

        lax.fori_loop(0, nbursts, body, 0)
        plsc.subcore_barrier()
        dst_off = pl.multiple_of(c * NPAD + off, 8)
        pltpu.sync_copy(acc.at[pl.ds(off, NPT)], out.at[pl.ds(dst_off, NPT)])

    return agg


def _row_spec(w):
    return pl.BlockSpec((BN, w), lambda i: (i, 0))


def _const_spec(shape):
    return pl.BlockSpec(shape, lambda i: (0, 0))


def _t1_body(x_ref, d0_ref, d1_ref, hs_ref, dv_ref):
    dv = lax.rsqrt(d0_ref[...] + d1_ref[...] + 1.0)
    dv_ref[...] = dv
    hs_ref[...] = x_ref[...] * dv


def _t2_body(a0, a1, hs, dv, w, b, out):
    z = (a0[...] + a1[...] + hs[...]) * dv[...]
    h = jnp.tanh(jnp.dot(z, w[...], preferred_element_type=_F32) + b[...])
    out[...] = h * dv[...]


def _l2(u):
    n = jnp.sqrt(jnp.sum(u * u, axis=1, keepdims=True))
    return u / jnp.maximum(n, 1e-12)


def _t3_body(a0, a1, hs, dv, w, b, out):
    z = (a0[...] + a1[...] + hs[...]) * dv[...]
    u = jnp.dot(z, w[...], preferred_element_type=_F32) + b[...]
    out[...] = jnp.tanh(_l2(u)) * dv[...]


def _t4_body(a0, a1, hs, dv, w3, b3, wc, bc, out):
    z = (a0[...] + a1[...] + hs[...]) * dv[...]
    u = jnp.dot(z, w3[...], preferred_element_type=_F32) + b3[...]
    h3 = _l2(u)
    v = jnp.dot(h3, wc[...], preferred_element_type=_F32) + bc[...]
    out[...] = _l2(v)


def _tc_call(body, in_arrays, widths, out_widths):
    grid = (NPAD // BN,)
    in_specs = []
    for a, w in zip(in_arrays, widths):
        if w is None:
            in_specs.append(_const_spec(a.shape))
        else:
            in_specs.append(_row_spec(w))
    out_shape = [jax.ShapeDtypeStruct((NPAD, w), _F32) for w in out_widths]
    out_specs = [_row_spec(w) for w in out_widths]
    if len(out_widths) == 1:
        out_shape, out_specs = out_shape[0], out_specs[0]
    return pl.pallas_call(
        body, grid=grid, in_specs=in_specs, out_specs=out_specs,
        out_shape=out_shape,
    )(*in_arrays)


def kernel(x, edge_index, W1, b1, W2, b2, W3, b3, Wc, bc):
    n = x.shape[0]
    e = edge_index.shape[1]
    burst_edges = 32 * K * CH
    nbursts = -(-e // burst_edges)
    epad = nbursts * burst_edges
    erows = epad // CH

    src = edge_index[0].astype(jnp.int32)
    dst = edge_index[1].astype(jnp.int32)
    pad = epad - e
    src2d = jnp.concatenate([src, jnp.zeros((pad,), jnp.int32)]).reshape(erows, CH)
    dst2d = jnp.concatenate([dst, jnp.full((pad,), n, jnp.int32)]).reshape(erows, CH)

    x16 = jnp.zeros((NPAD, 16), _F32).at[:n, :3].set(x)
    zrows = jnp.zeros((NPT, 16), _F32)
    zeros1 = jnp.zeros((NPT,), _F32)
    ones_h = jnp.ones((K, CH), _F32)

    w1p = jnp.zeros((16, 16), _F32).at[:3, :6].set(W1)
    b1p = jnp.zeros((1, 16), _F32).at[0, :6].set(b1)
    w2p = jnp.zeros((16, 16), _F32).at[:6, :12].set(W2)
    b2p = jnp.zeros((1, 16), _F32).at[0, :12].set(b2)
    w3p = jnp.zeros((16, 32), _F32).at[:12, :24].set(W3)
    b3p = jnp.zeros((1, 32), _F32).at[0, :24].set(b3)
    wcp = jnp.zeros((32, 16), _F32).at[:24, :13].set(Wc)
    bcp = jnp.zeros((1, 16), _F32).at[0, :13].set(bc)

    deg = _deg_kernel(nbursts)(dst2d, ones_h, zeros1)
    d0 = deg[:NPAD].reshape(NPAD, 1)
    d1 = deg[NPAD:].reshape(NPAD, 1)

    hs1, dv = _tc_call(_t1_body, [x16, d0, d1], [16, 1, 1], [16, 1])

    agg = _agg_kernel(nbursts)
    a1 = agg(hs1, src2d, dst2d, zrows)
    hs2 = _tc_call(
        _t2_body, [a1[:NPAD], a1[NPAD:], hs1, dv, w1p, b1p],
        [16, 16, 16, 1, None, None], [16])

    a2 = agg(hs2, src2d, dst2d, zrows)
    hs3 = _tc_call(
        _t3_body, [a2[:NPAD], a2[NPAD:], hs2, dv, w2p, b2p],
        [16, 16, 16, 1, None, None], [16])

    a3 = agg(hs3, src2d, dst2d, zrows)
    out16 = _tc_call(
        _t4_body, [a3[:NPAD], a3[NPAD:], hs3, dv, w3p, b3p, wcp, bcp],
        [16, 16, 16, 1, None, None, None, None], [16])

    return out16[:n, :13]


# speedup vs baseline: 39.6443x; 39.6443x over previous
import functools

import jax
import jax.numpy as jnp
from jax import lax
from jax.experimental import pallas as pl
from jax.experimental.pallas import tpu as pltpu
from jax.experimental.pallas import tpu_sc as plsc

NPAD = 100352
NPT = NPAD // 16
CH = 128
K = 8
BN = 2048
_F32 = jnp.float32

_MESH = plsc.VectorSubcoreMesh(core_axis_name="c", subcore_axis_name="s")
_SC_PARAMS = pltpu.CompilerParams(use_tc_tiling_on_sc=False)


def _deg_kernel(nbursts):
    @functools.partial(
        pl.kernel,
        out_type=jax.ShapeDtypeStruct((2 * NPAD,), _F32),
        mesh=_MESH,
        scratch_types=[
            pltpu.VMEM((K, CH), jnp.int32),
            pltpu.VMEM((K, CH), _F32),
            pltpu.VMEM_SHARED((NPAD,), _F32),
            pltpu.SemaphoreType.DMA,
        ],
        compiler_params=_SC_PARAMS,
    )
    def deg(dst2d, ones_h, zeros1, out, didx, ones_v, acc, sem):
        c = lax.axis_index("c")
        s = lax.axis_index("s")
        off = pl.multiple_of(s * NPT, 8)
        pltpu.sync_copy(zeros1, acc.at[pl.ds(off, NPT)])
        pltpu.sync_copy(ones_h, ones_v)
        plsc.subcore_barrier()
        row0 = (c * 16 + s) * (nbursts * K)

        def body(i, carry):
            rb = pl.multiple_of(row0 + i * K, 8)
            pltpu.sync_copy(dst2d.at[pl.ds(rb, K)], didx)
            descs = [
                pltpu.async_copy(ones_v.at[j], acc.at[didx.at[j]], sem, add=True)
                for j in range(K)
            ]
            for d in descs:
                d.wait()
            return carry

        lax.fori_loop(0, nbursts, body, 0)
        plsc.subcore_barrier()
        dst_off = pl.multiple_of(c * NPAD + off, 8)
        pltpu.sync_copy(acc.at[pl.ds(off, NPT)], out.at[pl.ds(dst_off, NPT)])

    return deg


def _agg_kernel(nbursts):
    @functools.partial(
        pl.kernel,
        out_type=jax.ShapeDtypeStruct((2 * NPAD, 16), _F32),
        mesh=_MESH,
        scratch_types=[
            pltpu.VMEM((K, CH), jnp.int32),
            pltpu.VMEM((K, CH), jnp.int32),
            pltpu.VMEM((K, CH, 16), _F32),
            pltpu.VMEM_SHARED((NPAD, 16), _F32),
            pltpu.SemaphoreType.DMA,
            pltpu.SemaphoreType.DMA,
        ],
        compiler_params=_SC_PARAMS,
    )
    def agg(table, src2d, dst2d, zrows, out, sidx, didx, rows, acc, sem_g, sem_s):
        c = lax.axis_index("c")
        s = lax.axis_index("s")
        off = pl.multiple_of(s * NPT, 8)
        pltpu.sync_copy(zrows, acc.at[pl.ds(off, NPT)])
        plsc.subcore_barrier()
        row0 = (c * 16 + s) * (nbursts * K)

        def body(i, carry):
            rb = pl.multiple_of(row0 + i * K, 8)
            pltpu.sync_copy(src2d.at[pl.ds(rb, K)], sidx)
            pltpu.sync_copy(dst2d.at[pl.ds(rb, K)], didx)
            gd = [
                pltpu.async_copy(table.at[sidx.at[j]], rows.at[j], sem_g)
                for j in range(K)
            ]
            for d in gd:
                d.wait()
            sd = [
                pltpu.async_copy(rows.at[j], acc.at[didx.at[j]], sem_s, add=True)
                for j in range(K)
            ]
            for d in sd:
                d.wait()
            return carry

        lax.fori_loop(0, nbursts, body, 0)
        plsc.subcore_barrier()
        dst_off = pl.multiple_of(c * NPAD + off, 8)
        pltpu.sync_copy(acc.at[pl.ds(off, NPT)], out.at[pl.ds(dst_off, NPT)])

    return agg


def _row_spec(w):
    return pl.BlockSpec((BN, w), lambda i: (i, 0))


def _const_spec(shape):
    return pl.BlockSpec(shape, lambda i: (0, 0))


def _t1_body(x_ref, d0_ref, d1_ref, hs_ref, dv_ref):
    dv = lax.rsqrt(d0_ref[...] + d1_ref[...] + 1.0)
    dv_ref[...] = dv
    hs_ref[...] = x_ref[...] * dv


def _t2_body(a0, a1, hs, dv, w, b, out):
    z = (a0[...] + a1[...] + hs[...]) * dv[...]
    h = jnp.tanh(jnp.dot(z, w[...], preferred_element_type=_F32, precision=lax.Precision.HIGHEST) + b[...])
    out[...] = h * dv[...]


def _l2(u):
    n = jnp.sqrt(jnp.sum(u * u, axis=1, keepdims=True))
    return u / jnp.maximum(n, 1e-12)


def _t3_body(a0, a1, hs, dv, w, b, out):
    z = (a0[...] + a1[...] + hs[...]) * dv[...]
    u = jnp.dot(z, w[...], preferred_element_type=_F32, precision=lax.Precision.HIGHEST) + b[...]
    out[...] = jnp.tanh(_l2(u)) * dv[...]


def _t4_body(a0, a1, hs, dv, w3, b3, wc, bc, out):
    z = (a0[...] + a1[...] + hs[...]) * dv[...]
    u = jnp.dot(z, w3[...], preferred_element_type=_F32, precision=lax.Precision.HIGHEST) + b3[...]
    h3 = _l2(u)
    v = jnp.dot(h3, wc[...], preferred_element_type=_F32, precision=lax.Precision.HIGHEST) + bc[...]
    out[...] = _l2(v)


def _tc_call(body, in_arrays, widths, out_widths):
    grid = (NPAD // BN,)
    in_specs = []
    for a, w in zip(in_arrays, widths):
        if w is None:
            in_specs.append(_const_spec(a.shape))
        else:
            in_specs.append(_row_spec(w))
    out_shape = [jax.ShapeDtypeStruct((NPAD, w), _F32) for w in out_widths]
    out_specs = [_row_spec(w) for w in out_widths]
    if len(out_widths) == 1:
        out_shape, out_specs = out_shape[0], out_specs[0]
    return pl.pallas_call(
        body, grid=grid, in_specs=in_specs, out_specs=out_specs,
        out_shape=out_shape,
    )(*in_arrays)


def kernel(x, edge_index, W1, b1, W2, b2, W3, b3, Wc, bc):
    n = x.shape[0]
    e = edge_index.shape[1]
    burst_edges = 32 * K * CH
    nbursts = -(-e // burst_edges)
    epad = nbursts * burst_edges
    erows = epad // CH

    src = edge_index[0].astype(jnp.int32)
    dst = edge_index[1].astype(jnp.int32)
    pad = epad - e
    src2d = jnp.concatenate([src, jnp.zeros((pad,), jnp.int32)]).reshape(erows, CH)
    dst2d = jnp.concatenate([dst, jnp.full((pad,), n, jnp.int32)]).reshape(erows, CH)

    x16 = jnp.zeros((NPAD, 16), _F32).at[:n, :3].set(x)
    zrows = jnp.zeros((NPT, 16), _F32)
    zeros1 = jnp.zeros((NPT,), _F32)
    ones_h = jnp.ones((K, CH), _F32)

    w1p = jnp.zeros((16, 16), _F32).at[:3, :6].set(W1)
    b1p = jnp.zeros((1, 16), _F32).at[0, :6].set(b1)
    w2p = jnp.zeros((16, 16), _F32).at[:6, :12].set(W2)
    b2p = jnp.zeros((1, 16), _F32).at[0, :12].set(b2)
    w3p = jnp.zeros((16, 32), _F32).at[:12, :24].set(W3)
    b3p = jnp.zeros((1, 32), _F32).at[0, :24].set(b3)
    wcp = jnp.zeros((32, 16), _F32).at[:24, :13].set(Wc)
    bcp = jnp.zeros((1, 16), _F32).at[0, :13].set(bc)

    deg = _deg_kernel(nbursts)(dst2d, ones_h, zeros1)
    d0 = deg[:NPAD].reshape(NPAD, 1)
    d1 = deg[NPAD:].reshape(NPAD, 1)

    hs1, dv = _tc_call(_t1_body, [x16, d0, d1], [16, 1, 1], [16, 1])

    agg = _agg_kernel(nbursts)
    a1 = agg(hs1, src2d, dst2d, zrows)
    hs2 = _tc_call(
        _t2_body, [a1[:NPAD], a1[NPAD:], hs1, dv, w1p, b1p],
        [16, 16, 16, 1, None, None], [16])

    a2 = agg(hs2, src2d, dst2d, zrows)
    hs3 = _tc_call(
        _t3_body, [a2[:NPAD], a2[NPAD:], hs2, dv, w2p, b2p],
        [16, 16, 16, 1, None, None], [16])

    a3 = agg(hs3, src2d, dst2d, zrows)
    out16 = _tc_call(
        _t4_body, [a3[:NPAD], a3[NPAD:], hs3, dv, w3p, b3p, wcp, bcp],
        [16, 16, 16, 1, None, None, None, None], [16])

    return out16[:n, :13]


# speedup vs baseline: 58.6746x; 1.4800x over previous
import functools

import jax
import jax.numpy as jnp
from jax import lax
from jax.experimental import pallas as pl
from jax.experimental.pallas import tpu as pltpu
from jax.experimental.pallas import tpu_sc as plsc

NPAD = 100352
NPT = NPAD // 16
NF = NPAD // 8
CH = 128
K = 8
BNODE = 2048
GRID = NPAD // BNODE
BR = 1792
_F32 = jnp.float32

_MESH = plsc.VectorSubcoreMesh(core_axis_name="c", subcore_axis_name="s")
_SC_PARAMS = pltpu.CompilerParams(use_tc_tiling_on_sc=False)


def _deg_kernel(nbursts):
    @functools.partial(
        pl.kernel,
        out_type=jax.ShapeDtypeStruct((2 * NPAD,), _F32),
        mesh=_MESH,
        scratch_types=[
            pltpu.VMEM((K, CH), jnp.int32),
            pltpu.VMEM((K, CH), _F32),
            pltpu.VMEM_SHARED((NPAD,), _F32),
            pltpu.SemaphoreType.DMA,
        ],
        compiler_params=_SC_PARAMS,
    )
    def deg(dst2d, ones_h, zeros1, out, didx, ones_v, acc, sem):
        c = lax.axis_index("c")
        s = lax.axis_index("s")
        off = pl.multiple_of(s * NPT, 8)
        pltpu.sync_copy(zeros1, acc.at[pl.ds(off, NPT)])
        pltpu.sync_copy(ones_h, ones_v)
        plsc.subcore_barrier()
        row0 = (c * 16 + s) * (nbursts * K)

        def body(i, carry):
            rb = pl.multiple_of(row0 + i * K, 8)
            pltpu.sync_copy(dst2d.at[pl.ds(rb, K)], didx)
            descs = [
                pltpu.async_copy(ones_v.at[j], acc.at[didx.at[j]], sem, add=True)
                for j in range(K)
            ]
            for d in descs:
                d.wait()
            return carry

        lax.fori_loop(0, nbursts, body, 0)
        plsc.subcore_barrier()
        dst_off = pl.multiple_of(c * NPAD + off, 8)
        pltpu.sync_copy(acc.at[pl.ds(off, NPT)], out.at[pl.ds(dst_off, NPT)])

    return deg


def _agg_kernel(nbursts):
    @functools.partial(
        pl.kernel,
        out_type=jax.ShapeDtypeStruct((2 * NPAD, 16), _F32),
        mesh=_MESH,
        scratch_types=[
            pltpu.VMEM((K, CH), jnp.int32),
            pltpu.VMEM((K, CH), jnp.int32),
            pltpu.VMEM((K, CH, 16), _F32),
            pltpu.VMEM_SHARED((NPAD, 16), _F32),
            pltpu.SemaphoreType.DMA,
            pltpu.SemaphoreType.DMA,
        ],
        compiler_params=_SC_PARAMS,
    )
    def agg(table, src2d, dst2d, zrows, out, sidx, didx, rows, acc, sem_g, sem_s):
        c = lax.axis_index("c")
        s = lax.axis_index("s")
        off = pl.multiple_of(s * NPT, 8)
        pltpu.sync_copy(zrows, acc.at[pl.ds(off, NPT)])
        plsc.subcore_barrier()
        row0 = (c * 16 + s) * (nbursts * K)

        def body(i, carry):
            rb = pl.multiple_of(row0 + i * K, 8)
            pltpu.sync_copy(src2d.at[pl.ds(rb, K)], sidx)
            pltpu.sync_copy(dst2d.at[pl.ds(rb, K)], didx)
            gd = [
                pltpu.async_copy(table.at[sidx.at[j]], rows.at[j], sem_g)
                for j in range(K)
            ]
            for d in gd:
                d.wait()
            sd = [
                pltpu.async_copy(rows.at[j], acc.at[didx.at[j]], sem_s, add=True)
                for j in range(K)
            ]
            for d in sd:
                d.wait()
            return carry

        lax.fori_loop(0, nbursts, body, 0)
        plsc.subcore_barrier()
        dst_off = pl.multiple_of(c * NPAD + off, 8)
        pltpu.sync_copy(acc.at[pl.ds(off, NPT)], out.at[pl.ds(dst_off, NPT)])

    return agg


def _t0_body(d0_ref, d1_ref, dv_ref):
    dv_ref[...] = lax.rsqrt(d0_ref[...] + d1_ref[...] + 1.0)


def _t1_body(x_ref, dvw_ref, hs_ref):
    hs_ref[...] = x_ref[...] * dvw_ref[...]


def _t2_body(a0, a1, hs, dvw, w, b, out):
    z = (a0[...] + a1[...] + hs[...]) * dvw[...]
    u = jnp.dot(z, w[...], preferred_element_type=_F32,
                precision=lax.Precision.HIGHEST) + b[...]
    out[...] = jnp.tanh(u) * dvw[...]


def _t3_body(a0, a1, hs, dvw, w, b, s16, out):
    z = (a0[...] + a1[...] + hs[...]) * dvw[...]
    u = jnp.dot(z, w[...], preferred_element_type=_F32,
                precision=lax.Precision.HIGHEST) + b[...]
    ss = jnp.dot(u * u, s16[...], preferred_element_type=_F32,
                 precision=lax.Precision.HIGHEST)
    h = jnp.tanh(u / jnp.maximum(jnp.sqrt(ss), 1e-12))
    out[...] = h * dvw[...]


def _t4_body(a0, a1, hs, dvw, w3l, w3r, b3l, b3r, wcl, wcr, bc, s16, out):
    hp = lax.Precision.HIGHEST
    z = (a0[...] + a1[...] + hs[...]) * dvw[...]
    ul = jnp.dot(z, w3l[...], preferred_element_type=_F32, precision=hp) + b3l[...]
    ur = jnp.dot(z, w3r[...], preferred_element_type=_F32, precision=hp) + b3r[...]
    ss = jnp.dot(ul * ul + ur * ur, s16[...], preferred_element_type=_F32,
                 precision=hp)
    inv = 1.0 / jnp.maximum(jnp.sqrt(ss), 1e-12)
    h3l = ul * inv
    h3r = ur * inv
    v = (jnp.dot(h3l, wcl[...], preferred_element_type=_F32, precision=hp)
         + jnp.dot(h3r, wcr[...], preferred_element_type=_F32, precision=hp)
         + bc[...])
    ss2 = jnp.dot(v * v, s16[...], preferred_element_type=_F32, precision=hp)
    out[...] = v / jnp.maximum(jnp.sqrt(ss2), 1e-12)


def _spec(rows, imap):
    return pl.BlockSpec((rows, 128), imap)


def _cspec(shape):
    return pl.BlockSpec(shape, lambda i: (0, 0))


def _blockdiag(w16):
    return jnp.kron(jnp.eye(8, dtype=_F32), w16)


def kernel(x, edge_index, W1, b1, W2, b2, W3, b3, Wc, bc):
    n = x.shape[0]
    e = edge_index.shape[1]
    burst_edges = 32 * K * CH
    nbursts = -(-e // burst_edges)
    epad = nbursts * burst_edges
    erows = epad // CH

    src = edge_index[0].astype(jnp.int32)
    dst = edge_index[1].astype(jnp.int32)
    pad = epad - e
    src2d = jnp.concatenate([src, jnp.zeros((pad,), jnp.int32)]).reshape(erows, CH)
    dst2d = jnp.concatenate([dst, jnp.full((pad,), n, jnp.int32)]).reshape(erows, CH)

    zrows = jnp.zeros((NPT, 16), _F32)
    zeros1 = jnp.zeros((NPT,), _F32)
    ones_h = jnp.ones((K, CH), _F32)

    w1b = _blockdiag(jnp.zeros((16, 16), _F32).at[:3, :6].set(W1))
    b1b = jnp.tile(jnp.zeros((1, 16), _F32).at[0, :6].set(b1), (1, 8))
    w2b = _blockdiag(jnp.zeros((16, 16), _F32).at[:6, :12].set(W2))
    b2b = jnp.tile(jnp.zeros((1, 16), _F32).at[0, :12].set(b2), (1, 8))
    w3lb = _blockdiag(jnp.zeros((16, 16), _F32).at[:12, :12].set(W3[:, :12]))
    w3rb = _blockdiag(jnp.zeros((16, 16), _F32).at[:12, :12].set(W3[:, 12:]))
    b3lb = jnp.tile(jnp.zeros((1, 16), _F32).at[0, :12].set(b3[:12]), (1, 8))
    b3rb = jnp.tile(jnp.zeros((1, 16), _F32).at[0, :12].set(b3[12:]), (1, 8))
    wclb = _blockdiag(jnp.zeros((16, 16), _F32).at[:12, :13].set(Wc[:12]))
    wcrb = _blockdiag(jnp.zeros((16, 16), _F32).at[:12, :13].set(Wc[12:]))
    bcb = jnp.tile(jnp.zeros((1, 16), _F32).at[0, :13].set(bc), (1, 8))
    s16b = _blockdiag(jnp.ones((16, 16), _F32))

    deg = _deg_kernel(nbursts)(dst2d, ones_h, zeros1)
    deg2d = deg.reshape(2 * NPAD // 128, 128)

    dv_lanes = pl.pallas_call(
        _t0_body, grid=(GRID,),
        in_specs=[
            pl.BlockSpec((16, 128), lambda i: (i, 0)),
            pl.BlockSpec((16, 128), lambda i: (i + GRID, 0)),
        ],
        out_specs=pl.BlockSpec((16, 128), lambda i: (i, 0)),
        out_shape=jax.ShapeDtypeStruct((NPAD // 128, 128), _F32),
    )(deg2d, deg2d)

    dvwf = jnp.broadcast_to(dv_lanes.reshape(NPAD, 1), (NPAD, 16)).reshape(NF, 128)
    x16f = jnp.zeros((NPAD, 16), _F32).at[:n, :3].set(x).reshape(NF, 128)

    hs1f = pl.pallas_call(
        _t1_body, grid=(NF // BR,),
        in_specs=[_spec(BR, lambda i: (i, 0))] * 2,
        out_specs=_spec(BR, lambda i: (i, 0)),
        out_shape=jax.ShapeDtypeStruct((NF, 128), _F32),
    )(x16f, dvwf)

    agg = _agg_kernel(nbursts)
    nfb = NF // BR

    def dense(body, aggf, hsf, consts):
        cspecs = [_cspec(c.shape) for c in consts]
        return pl.pallas_call(
            body, grid=(nfb,),
            in_specs=[
                _spec(BR, lambda i: (i, 0)),
                _spec(BR, lambda i: (i + nfb, 0)),
                _spec(BR, lambda i: (i, 0)),
                _spec(BR, lambda i: (i, 0)),
            ] + cspecs,
            out_specs=_spec(BR, lambda i: (i, 0)),
            out_shape=jax.ShapeDtypeStruct((NF, 128), _F32),
        )(aggf, aggf, hsf, dvwf, *consts)

    a1f = agg(hs1f.reshape(NPAD, 16), src2d, dst2d, zrows).reshape(2 * NF, 128)
    hs2f = dense(_t2_body, a1f, hs1f, [w1b, b1b])

    a2f = agg(hs2f.reshape(NPAD, 16), src2d, dst2d, zrows).reshape(2 * NF, 128)
    hs3f = dense(_t3_body, a2f, hs2f, [w2b, b2b, s16b])

    a3f = agg(hs3f.reshape(NPAD, 16), src2d, dst2d, zrows).reshape(2 * NF, 128)
    outf = dense(_t4_body, a3f, hs3f,
                 [w3lb, w3rb, b3lb, b3rb, wclb, wcrb, bcb, s16b])

    return outf.reshape(NPAD, 16)[:n, :13]


# speedup vs baseline: 66.4979x; 1.1333x over previous
import functools

import jax
import jax.numpy as jnp
from jax import lax
from jax.experimental import pallas as pl
from jax.experimental.pallas import tpu as pltpu
from jax.experimental.pallas import tpu_sc as plsc

NPAD = 100352
NPT = NPAD // 16
NF = NPAD // 8
CH = 128
K = 4
BNODE = 2048
GRID = NPAD // BNODE
BR = 1792
_F32 = jnp.float32

_MESH = plsc.VectorSubcoreMesh(core_axis_name="c", subcore_axis_name="s")
_SC_PARAMS = pltpu.CompilerParams(use_tc_tiling_on_sc=False)


def _deg_kernel(nbursts):
    @functools.partial(
        pl.kernel,
        out_type=jax.ShapeDtypeStruct((2 * NPAD,), _F32),
        mesh=_MESH,
        scratch_types=[
            pltpu.VMEM((K, CH), jnp.int32),
            pltpu.VMEM((K, CH), _F32),
            pltpu.VMEM_SHARED((NPAD,), _F32),
            pltpu.SemaphoreType.DMA,
        ],
        compiler_params=_SC_PARAMS,
    )
    def deg(dst2d, ones_h, zeros1, out, didx, ones_v, acc, sem):
        c = lax.axis_index("c")
        s = lax.axis_index("s")
        off = pl.multiple_of(s * NPT, 8)
        pltpu.sync_copy(zeros1, acc.at[pl.ds(off, NPT)])
        pltpu.sync_copy(ones_h, ones_v)
        plsc.subcore_barrier()
        row0 = (c * 16 + s) * (nbursts * K)

        def body(i, carry):
            rb = pl.multiple_of(row0 + i * K, 4)
            pltpu.sync_copy(dst2d.at[pl.ds(rb, K)], didx)
            descs = [
                pltpu.async_copy(ones_v.at[j], acc.at[didx.at[j]], sem, add=True)
                for j in range(K)
            ]
            for d in descs:
                d.wait()
            return carry

        lax.fori_loop(0, nbursts, body, 0)
        plsc.subcore_barrier()
        dst_off = pl.multiple_of(c * NPAD + off, 8)
        pltpu.sync_copy(acc.at[pl.ds(off, NPT)], out.at[pl.ds(dst_off, NPT)])

    return deg


def _agg_kernel(nbursts):
    assert nbursts % 2 == 0
    npairs = nbursts // 2

    @functools.partial(
        pl.kernel,
        out_type=jax.ShapeDtypeStruct((2 * NPAD, 16), _F32),
        mesh=_MESH,
        scratch_types=[
            pltpu.VMEM((2, K, CH), jnp.int32),
            pltpu.VMEM((2, K, CH), jnp.int32),
            pltpu.VMEM((2, K, CH, 16), _F32),
            pltpu.VMEM_SHARED((NPAD, 16), _F32),
            pltpu.SemaphoreType.DMA,
            pltpu.SemaphoreType.DMA,
            pltpu.SemaphoreType.DMA,
            pltpu.SemaphoreType.DMA,
        ],
        compiler_params=_SC_PARAMS,
    )
    def agg(table, src2d, dst2d, zrows, out, sidx, didx, rows, acc,
            sem_g0, sem_g1, sem_s0, sem_s1):
        c = lax.axis_index("c")
        s = lax.axis_index("s")
        off = pl.multiple_of(s * NPT, 8)
        pltpu.sync_copy(zrows, acc.at[pl.ds(off, NPT)])
        plsc.subcore_barrier()
        row0 = (c * 16 + s) * (nbursts * K)
        sem_g = (sem_g0, sem_g1)
        sem_s = (sem_s0, sem_s1)

        def fire_gathers(i, b):
            rb = pl.multiple_of(row0 + i * K, 4)
            pltpu.sync_copy(src2d.at[pl.ds(rb, K)], sidx.at[b])
            pltpu.sync_copy(dst2d.at[pl.ds(rb, K)], didx.at[b])
            for j in range(K):
                pltpu.async_copy(table.at[sidx.at[b, j]], rows.at[b, j], sem_g[b])

        def drain_gathers(b):
            for j in range(K):
                pltpu.make_async_copy(
                    table.at[sidx.at[b, j]], rows.at[b, j], sem_g[b]).wait()

        def fire_scatters(b):
            for j in range(K):
                pltpu.async_copy(
                    rows.at[b, j], acc.at[didx.at[b, j]], sem_s[b], add=True)

        def drain_scatters(b):
            for j in range(K):
                pltpu.make_async_copy(
                    rows.at[b, j], acc.at[didx.at[b, j]], sem_s[b]).wait()

        fire_gathers(0, 0)

        def body(t, carry):
            a = 2 * t

            @pl.when(t > 0)
            def _():
                drain_scatters(1)
            fire_gathers(a + 1, 1)
            drain_gathers(0)
            fire_scatters(0)
            drain_scatters(0)

            @pl.when(t + 1 < npairs)
            def _():
                fire_gathers(a + 2, 0)
            drain_gathers(1)
            fire_scatters(1)
            return carry

        lax.fori_loop(0, npairs, body, 0)
        drain_scatters(1)
        plsc.subcore_barrier()
        dst_off = pl.multiple_of(c * NPAD + off, 8)
        pltpu.sync_copy(acc.at[pl.ds(off, NPT)], out.at[pl.ds(dst_off, NPT)])

    return agg


def _t0_body(d0_ref, d1_ref, dv_ref):
    dv_ref[...] = lax.rsqrt(d0_ref[...] + d1_ref[...] + 1.0)


def _t1_body(x_ref, dvw_ref, hs_ref):
    hs_ref[...] = x_ref[...] * dvw_ref[...]


def _t2_body(a0, a1, hs, dvw, w, b, out):
    z = (a0[...] + a1[...] + hs[...]) * dvw[...]
    u = jnp.dot(z, w[...], preferred_element_type=_F32,
                precision=lax.Precision.HIGHEST) + b[...]
    out[...] = jnp.tanh(u) * dvw[...]


def _t3_body(a0, a1, hs, dvw, w, b, s16, out):
    z = (a0[...] + a1[...] + hs[...]) * dvw[...]
    u = jnp.dot(z, w[...], preferred_element_type=_F32,
                precision=lax.Precision.HIGHEST) + b[...]
    ss = jnp.dot(u * u, s16[...], preferred_element_type=_F32,
                 precision=lax.Precision.HIGHEST)
    h = jnp.tanh(u / jnp.maximum(jnp.sqrt(ss), 1e-12))
    out[...] = h * dvw[...]


def _t4_body(a0, a1, hs, dvw, w3l, w3r, b3l, b3r, wcl, wcr, bc, s16, out):
    hp = lax.Precision.HIGHEST
    z = (a0[...] + a1[...] + hs[...]) * dvw[...]
    ul = jnp.dot(z, w3l[...], preferred_element_type=_F32, precision=hp) + b3l[...]
    ur = jnp.dot(z, w3r[...], preferred_element_type=_F32, precision=hp) + b3r[...]
    ss = jnp.dot(ul * ul + ur * ur, s16[...], preferred_element_type=_F32,
                 precision=hp)
    inv = 1.0 / jnp.maximum(jnp.sqrt(ss), 1e-12)
    h3l = ul * inv
    h3r = ur * inv
    v = (jnp.dot(h3l, wcl[...], preferred_element_type=_F32, precision=hp)
         + jnp.dot(h3r, wcr[...], preferred_element_type=_F32, precision=hp)
         + bc[...])
    ss2 = jnp.dot(v * v, s16[...], preferred_element_type=_F32, precision=hp)
    out[...] = v / jnp.maximum(jnp.sqrt(ss2), 1e-12)


def _spec(rows, imap):
    return pl.BlockSpec((rows, 128), imap)


def _cspec(shape):
    return pl.BlockSpec(shape, lambda i: (0, 0))


def _blockdiag(w16):
    return jnp.kron(jnp.eye(8, dtype=_F32), w16)


def kernel(x, edge_index, W1, b1, W2, b2, W3, b3, Wc, bc):
    n = x.shape[0]
    e = edge_index.shape[1]
    burst_edges = 32 * K * CH
    nbursts = -(-e // burst_edges)
    nbursts += nbursts % 2
    epad = nbursts * burst_edges
    erows = epad // CH

    src = edge_index[0].astype(jnp.int32)
    dst = edge_index[1].astype(jnp.int32)
    pad = epad - e
    src2d = jnp.concatenate([src, jnp.zeros((pad,), jnp.int32)]).reshape(erows, CH)
    dst2d = jnp.concatenate([dst, jnp.full((pad,), n, jnp.int32)]).reshape(erows, CH)

    zrows = jnp.zeros((NPT, 16), _F32)
    zeros1 = jnp.zeros((NPT,), _F32)
    ones_h = jnp.ones((K, CH), _F32)

    w1b = _blockdiag(jnp.zeros((16, 16), _F32).at[:3, :6].set(W1))
    b1b = jnp.tile(jnp.zeros((1, 16), _F32).at[0, :6].set(b1), (1, 8))
    w2b = _blockdiag(jnp.zeros((16, 16), _F32).at[:6, :12].set(W2))
    b2b = jnp.tile(jnp.zeros((1, 16), _F32).at[0, :12].set(b2), (1, 8))
    w3lb = _blockdiag(jnp.zeros((16, 16), _F32).at[:12, :12].set(W3[:, :12]))
    w3rb = _blockdiag(jnp.zeros((16, 16), _F32).at[:12, :12].set(W3[:, 12:]))
    b3lb = jnp.tile(jnp.zeros((1, 16), _F32).at[0, :12].set(b3[:12]), (1, 8))
    b3rb = jnp.tile(jnp.zeros((1, 16), _F32).at[0, :12].set(b3[12:]), (1, 8))
    wclb = _blockdiag(jnp.zeros((16, 16), _F32).at[:12, :13].set(Wc[:12]))
    wcrb = _blockdiag(jnp.zeros((16, 16), _F32).at[:12, :13].set(Wc[12:]))
    bcb = jnp.tile(jnp.zeros((1, 16), _F32).at[0, :13].set(bc), (1, 8))
    s16b = _blockdiag(jnp.ones((16, 16), _F32))

    deg = _deg_kernel(nbursts)(dst2d, ones_h, zeros1)
    deg2d = deg.reshape(2 * NPAD // 128, 128)

    dv_lanes = pl.pallas_call(
        _t0_body, grid=(GRID,),
        in_specs=[
            pl.BlockSpec((16, 128), lambda i: (i, 0)),
            pl.BlockSpec((16, 128), lambda i: (i + GRID, 0)),
        ],
        out_specs=pl.BlockSpec((16, 128), lambda i: (i, 0)),
        out_shape=jax.ShapeDtypeStruct((NPAD // 128, 128), _F32),
    )(deg2d, deg2d)

    dvwf = jnp.broadcast_to(dv_lanes.reshape(NPAD, 1), (NPAD, 16)).reshape(NF, 128)
    x16f = jnp.zeros((NPAD, 16), _F32).at[:n, :3].set(x).reshape(NF, 128)

    hs1f = pl.pallas_call(
        _t1_body, grid=(NF // BR,),
        in_specs=[_spec(BR, lambda i: (i, 0))] * 2,
        out_specs=_spec(BR, lambda i: (i, 0)),
        out_shape=jax.ShapeDtypeStruct((NF, 128), _F32),
    )(x16f, dvwf)

    agg = _agg_kernel(nbursts)
    nfb = NF // BR

    def dense(body, aggf, hsf, consts):
        cspecs = [_cspec(c.shape) for c in consts]
        return pl.pallas_call(
            body, grid=(nfb,),
            in_specs=[
                _spec(BR, lambda i: (i, 0)),
                _spec(BR, lambda i: (i + nfb, 0)),
                _spec(BR, lambda i: (i, 0)),
                _spec(BR, lambda i: (i, 0)),
            ] + cspecs,
            out_specs=_spec(BR, lambda i: (i, 0)),
            out_shape=jax.ShapeDtypeStruct((NF, 128), _F32),
        )(aggf, aggf, hsf, dvwf, *consts)

    a1f = agg(hs1f.reshape(NPAD, 16), src2d, dst2d, zrows).reshape(2 * NF, 128)
    hs2f = dense(_t2_body, a1f, hs1f, [w1b, b1b])

    a2f = agg(hs2f.reshape(NPAD, 16), src2d, dst2d, zrows).reshape(2 * NF, 128)
    hs3f = dense(_t3_body, a2f, hs2f, [w2b, b2b, s16b])

    a3f = agg(hs3f.reshape(NPAD, 16), src2d, dst2d, zrows).reshape(2 * NF, 128)
    outf = dense(_t4_body, a3f, hs3f,
                 [w3lb, w3rb, b3lb, b3rb, wclb, wcrb, bcb, s16b])

    return outf.reshape(NPAD, 16)[:n, :13]


# speedup vs baseline: 76.5261x; 1.1508x over previous
import functools

import jax
import jax.numpy as jnp
from jax import lax
from jax.experimental import pallas as pl
from jax.experimental.pallas import tpu as pltpu
from jax.experimental.pallas import tpu_sc as plsc

NPAD = 100352
NPT = NPAD // 16
NF = NPAD // 8
CH = 128
K = 4
BNODE = 2048
GRID = NPAD // BNODE
BR = 1792
_F32 = jnp.float32

_MESH = plsc.VectorSubcoreMesh(core_axis_name="c", subcore_axis_name="s")
_SC_PARAMS = pltpu.CompilerParams(use_tc_tiling_on_sc=False)


KD = 8


def _deg_kernel(nbursts):
    @functools.partial(
        pl.kernel,
        out_type=jax.ShapeDtypeStruct((2 * NPAD,), _F32),
        mesh=_MESH,
        scratch_types=[
            pltpu.VMEM((KD, CH), jnp.int32),
            pltpu.VMEM((KD, CH), _F32),
            pltpu.VMEM_SHARED((NPAD,), _F32),
            pltpu.SemaphoreType.DMA,
        ],
        compiler_params=_SC_PARAMS,
    )
    def deg(dst2d, ones_h, zeros1, out, didx, ones_v, acc, sem):
        c = lax.axis_index("c")
        s = lax.axis_index("s")
        off = pl.multiple_of(s * NPT, 8)
        pltpu.sync_copy(zeros1, acc.at[pl.ds(off, NPT)])
        pltpu.sync_copy(ones_h, ones_v)
        plsc.subcore_barrier()
        row0 = (c * 16 + s) * (nbursts * KD)

        def body(i, carry):
            rb = pl.multiple_of(row0 + i * KD, 8)
            pltpu.sync_copy(dst2d.at[pl.ds(rb, KD)], didx)
            descs = [
                pltpu.async_copy(ones_v.at[j], acc.at[didx.at[j]], sem, add=True)
                for j in range(KD)
            ]
            for d in descs:
                d.wait()
            return carry

        lax.fori_loop(0, nbursts, body, 0)
        plsc.subcore_barrier()
        dst_off = pl.multiple_of(c * NPAD + off, 8)
        pltpu.sync_copy(acc.at[pl.ds(off, NPT)], out.at[pl.ds(dst_off, NPT)])

    return deg


def _agg_kernel(nbursts):
    assert nbursts % 2 == 0
    npairs = nbursts // 2

    @functools.partial(
        pl.kernel,
        out_type=jax.ShapeDtypeStruct((2 * NPAD, 16), _F32),
        mesh=_MESH,
        scratch_types=[
            pltpu.VMEM((2, 2 * K, CH), jnp.int32),
            pltpu.VMEM((2, K, CH, 16), _F32),
            pltpu.VMEM_SHARED((NPAD, 16), _F32),
            pltpu.SemaphoreType.DMA,
            pltpu.SemaphoreType.DMA,
            pltpu.SemaphoreType.DMA,
            pltpu.SemaphoreType.DMA,
        ],
        compiler_params=_SC_PARAMS,
    )
    def agg(table, eidx2d, zrows, out, eidx, rows, acc,
            sem_g0, sem_g1, sem_s0, sem_s1):
        c = lax.axis_index("c")
        s = lax.axis_index("s")
        off = pl.multiple_of(s * NPT, 8)
        pltpu.sync_copy(zrows, acc.at[pl.ds(off, NPT)])
        plsc.subcore_barrier()
        row0 = (c * 16 + s) * (nbursts * K)
        sem_g = (sem_g0, sem_g1)
        sem_s = (sem_s0, sem_s1)

        def fire_gathers(i, b):
            rb = pl.multiple_of(2 * (row0 + i * K), 8)
            pltpu.sync_copy(eidx2d.at[pl.ds(rb, 2 * K)], eidx.at[b])
            for j in range(K):
                pltpu.async_copy(
                    table.at[eidx.at[b, 2 * j]], rows.at[b, j], sem_g[b])

        def drain_gathers(b):
            for j in range(K):
                pltpu.make_async_copy(
                    table.at[eidx.at[b, 2 * j]], rows.at[b, j], sem_g[b]).wait()

        def fire_scatters(b):
            for j in range(K):
                pltpu.async_copy(
                    rows.at[b, j], acc.at[eidx.at[b, 2 * j + 1]], sem_s[b],
                    add=True)

        def drain_scatters(b):
            for j in range(K):
                pltpu.make_async_copy(
                    rows.at[b, j], acc.at[eidx.at[b, 2 * j + 1]], sem_s[b]).wait()

        fire_gathers(0, 0)

        def body(t, carry):
            a = 2 * t

            @pl.when(t > 0)
            def _():
                drain_scatters(1)
            fire_gathers(a + 1, 1)
            drain_gathers(0)
            fire_scatters(0)
            drain_scatters(0)

            @pl.when(t + 1 < npairs)
            def _():
                fire_gathers(a + 2, 0)
            drain_gathers(1)
            fire_scatters(1)
            return carry

        lax.fori_loop(0, npairs, body, 0)
        drain_scatters(1)
        plsc.subcore_barrier()
        dst_off = pl.multiple_of(c * NPAD + off, 8)
        pltpu.sync_copy(acc.at[pl.ds(off, NPT)], out.at[pl.ds(dst_off, NPT)])

    return agg


def _t0_body(d0_ref, d1_ref, dv_ref):
    dv_ref[...] = lax.rsqrt(d0_ref[...] + d1_ref[...] + 1.0)


def _t1_body(x_ref, dvw_ref, hs_ref):
    hs_ref[...] = x_ref[...] * dvw_ref[...]


def _t2_body(a0, a1, hs, dvw, w, b, out):
    z = (a0[...] + a1[...] + hs[...]) * dvw[...]
    u = jnp.dot(z, w[...], preferred_element_type=_F32,
                precision=lax.Precision.HIGHEST) + b[...]
    out[...] = jnp.tanh(u) * dvw[...]


def _t3_body(a0, a1, hs, dvw, w, b, s16, out):
    z = (a0[...] + a1[...] + hs[...]) * dvw[...]
    u = jnp.dot(z, w[...], preferred_element_type=_F32,
                precision=lax.Precision.HIGHEST) + b[...]
    ss = jnp.dot(u * u, s16[...], preferred_element_type=_F32,
                 precision=lax.Precision.HIGHEST)
    h = jnp.tanh(u / jnp.maximum(jnp.sqrt(ss), 1e-12))
    out[...] = h * dvw[...]


def _t4_body(a0, a1, hs, dvw, w3l, w3r, b3l, b3r, wcl, wcr, bc, s16, out):
    hp = lax.Precision.HIGHEST
    z = (a0[...] + a1[...] + hs[...]) * dvw[...]
    ul = jnp.dot(z, w3l[...], preferred_element_type=_F32, precision=hp) + b3l[...]
    ur = jnp.dot(z, w3r[...], preferred_element_type=_F32, precision=hp) + b3r[...]
    ss = jnp.dot(ul * ul + ur * ur, s16[...], preferred_element_type=_F32,
                 precision=hp)
    inv = 1.0 / jnp.maximum(jnp.sqrt(ss), 1e-12)
    h3l = ul * inv
    h3r = ur * inv
    v = (jnp.dot(h3l, wcl[...], preferred_element_type=_F32, precision=hp)
         + jnp.dot(h3r, wcr[...], preferred_element_type=_F32, precision=hp)
         + bc[...])
    ss2 = jnp.dot(v * v, s16[...], preferred_element_type=_F32, precision=hp)
    out[...] = v / jnp.maximum(jnp.sqrt(ss2), 1e-12)


def _spec(rows, imap):
    return pl.BlockSpec((rows, 128), imap)


def _cspec(shape):
    return pl.BlockSpec(shape, lambda i: (0, 0))


def _blockdiag(w16):
    return jnp.kron(jnp.eye(8, dtype=_F32), w16)


def kernel(x, edge_index, W1, b1, W2, b2, W3, b3, Wc, bc):
    n = x.shape[0]
    e = edge_index.shape[1]
    burst_edges = 32 * K * CH
    nbursts = -(-e // burst_edges)
    nbursts += nbursts % 2
    epad = nbursts * burst_edges
    erows = epad // CH
    nbursts_d = epad // (32 * KD * CH)
    assert nbursts_d * 32 * KD * CH == epad

    src = edge_index[0].astype(jnp.int32)
    dst = edge_index[1].astype(jnp.int32)
    pad = epad - e
    src2d = jnp.concatenate([src, jnp.zeros((pad,), jnp.int32)]).reshape(erows, CH)
    dst2d = jnp.concatenate([dst, jnp.full((pad,), n, jnp.int32)]).reshape(erows, CH)

    zrows = jnp.zeros((NPT, 16), _F32)
    zeros1 = jnp.zeros((NPT,), _F32)
    ones_h = jnp.ones((KD, CH), _F32)

    w1b = _blockdiag(jnp.zeros((16, 16), _F32).at[:3, :6].set(W1))
    b1b = jnp.tile(jnp.zeros((1, 16), _F32).at[0, :6].set(b1), (1, 8))
    w2b = _blockdiag(jnp.zeros((16, 16), _F32).at[:6, :12].set(W2))
    b2b = jnp.tile(jnp.zeros((1, 16), _F32).at[0, :12].set(b2), (1, 8))
    w3lb = _blockdiag(jnp.zeros((16, 16), _F32).at[:12, :12].set(W3[:, :12]))
    w3rb = _blockdiag(jnp.zeros((16, 16), _F32).at[:12, :12].set(W3[:, 12:]))
    b3lb = jnp.tile(jnp.zeros((1, 16), _F32).at[0, :12].set(b3[:12]), (1, 8))
    b3rb = jnp.tile(jnp.zeros((1, 16), _F32).at[0, :12].set(b3[12:]), (1, 8))
    wclb = _blockdiag(jnp.zeros((16, 16), _F32).at[:12, :13].set(Wc[:12]))
    wcrb = _blockdiag(jnp.zeros((16, 16), _F32).at[:12, :13].set(Wc[12:]))
    bcb = jnp.tile(jnp.zeros((1, 16), _F32).at[0, :13].set(bc), (1, 8))
    s16b = _blockdiag(jnp.ones((16, 16), _F32))

    deg = _deg_kernel(nbursts_d)(dst2d, ones_h, zeros1)
    deg2d = deg.reshape(2 * NPAD // 128, 128)

    dv_lanes = pl.pallas_call(
        _t0_body, grid=(GRID,),
        in_specs=[
            pl.BlockSpec((16, 128), lambda i: (i, 0)),
            pl.BlockSpec((16, 128), lambda i: (i + GRID, 0)),
        ],
        out_specs=pl.BlockSpec((16, 128), lambda i: (i, 0)),
        out_shape=jax.ShapeDtypeStruct((NPAD // 128, 128), _F32),
    )(deg2d, deg2d)

    dvwf = jnp.broadcast_to(dv_lanes.reshape(NPAD, 1), (NPAD, 16)).reshape(NF, 128)
    x16f = jnp.zeros((NPAD, 16), _F32).at[:n, :3].set(x).reshape(NF, 128)

    hs1f = pl.pallas_call(
        _t1_body, grid=(NF // BR,),
        in_specs=[_spec(BR, lambda i: (i, 0))] * 2,
        out_specs=_spec(BR, lambda i: (i, 0)),
        out_shape=jax.ShapeDtypeStruct((NF, 128), _F32),
    )(x16f, dvwf)

    agg = _agg_kernel(nbursts)
    nfb = NF // BR

    def dense(body, aggf, hsf, consts):
        cspecs = [_cspec(c.shape) for c in consts]
        return pl.pallas_call(
            body, grid=(nfb,),
            in_specs=[
                _spec(BR, lambda i: (i, 0)),
                _spec(BR, lambda i: (i + nfb, 0)),
                _spec(BR, lambda i: (i, 0)),
                _spec(BR, lambda i: (i, 0)),
            ] + cspecs,
            out_specs=_spec(BR, lambda i: (i, 0)),
            out_shape=jax.ShapeDtypeStruct((NF, 128), _F32),
        )(aggf, aggf, hsf, dvwf, *consts)

    eidx2d = jnp.stack([src2d, dst2d], axis=1).reshape(2 * erows, CH)

    a1f = agg(hs1f.reshape(NPAD, 16), eidx2d, zrows).reshape(2 * NF, 128)
    hs2f = dense(_t2_body, a1f, hs1f, [w1b, b1b])

    a2f = agg(hs2f.reshape(NPAD, 16), eidx2d, zrows).reshape(2 * NF, 128)
    hs3f = dense(_t3_body, a2f, hs2f, [w2b, b2b, s16b])

    a3f = agg(hs3f.reshape(NPAD, 16), eidx2d, zrows).reshape(2 * NF, 128)
    outf = dense(_t4_body, a3f, hs3f,
                 [w3lb, w3rb, b3lb, b3rb, wclb, wcrb, bcb, s16b])

    return outf.reshape(NPAD, 16)[:n, :13]
